# Initial kernel scaffold; baseline (speedup 1.0000x reference)
import jax
import jax.numpy as jnp
from jax.experimental import pallas as pl


def kernel(x, edge_index, edge_attr, batch, We1, be1, W1a, b1a, W1b, b1b, We2, be2, W2a, b2a, W2b, b2b, Wc, bc):
    raise NotImplementedError("write your pallas kernel here")


# speedup vs baseline: 1.5781x; 1.5781x over previous
import functools

import jax
import jax.numpy as jnp
from jax import lax
from jax.experimental import pallas as pl
from jax.experimental.pallas import tpu as pltpu
from jax.experimental.pallas import tpu_sc as plsc

NC, NS = 2, 16
NW = NC * NS
FC = 128
WIN = 40
POOL_B = 64

_MESH = dict(core_axis_name="c", subcore_axis_name="s",
             num_cores=NC, num_subcores=NS)


def _sc_gather(table, idx):
    n, f = table.shape
    e = idx.shape[0]
    per_w = e // NW
    nwin = per_w // WIN

    @functools.partial(
        pl.kernel,
        mesh=plsc.VectorSubcoreMesh(**_MESH),
        out_type=jax.ShapeDtypeStruct((e, f), jnp.float32),
        scratch_types=[
            pltpu.VMEM((per_w,), jnp.int32),
            pltpu.VMEM((WIN, f), jnp.float32),
            pltpu.SemaphoreType.DMA,
        ],
    )
    def k(table_hbm, idx_hbm, out_hbm, idx_v, rows_v, sem):
        wid = lax.axis_index("s") * NC + lax.axis_index("c")
        base = pl.multiple_of(wid * per_w, per_w)
        pltpu.sync_copy(idx_hbm.at[pl.ds(base, per_w)], idx_v)

        def body(i, carry):
            off = pl.multiple_of(i * WIN, WIN)
            pltpu.async_copy(
                table_hbm.at[idx_v.at[pl.ds(off, WIN)]], rows_v, sem
            ).wait()
            pltpu.sync_copy(
                rows_v, out_hbm.at[pl.ds(pl.multiple_of(base + off, WIN), WIN)]
            )
            return carry

        lax.fori_loop(0, nwin, body, 0)

    return k(table, idx)


def _sc_scatter_add(m, dst_w, h, np_rows):
    nq, e, _ = m.shape
    n, f = h.shape
    rows_t = np_rows // NS
    rem = n - (NS - 1) * rows_t
    e_t = e // NS
    wins_t = e_t // WIN
    passes = nq // NC

    @functools.partial(
        pl.kernel,
        mesh=plsc.VectorSubcoreMesh(**_MESH),
        out_type=jax.ShapeDtypeStruct((np_rows, nq * FC), jnp.float32),
        scratch_types=[
            pltpu.VMEM((wins_t, WIN), jnp.int32),
            pltpu.VMEM((WIN, FC), jnp.float32),
            pltpu.VMEM_SHARED((np_rows, FC), jnp.float32),
            pltpu.SemaphoreType.DMA,
        ],
    )
    def k(m_hbm, dst_hbm, h_hbm, out_hbm, idx_v, mv, acc, sem):
        cid = lax.axis_index("c")
        sid = lax.axis_index("s")
        row0 = pl.multiple_of(sid * rows_t, rows_t)
        pltpu.sync_copy(dst_hbm.at[sid], idx_v)

        for p in range(passes):
            q = p * NC + cid
            col0 = pl.multiple_of(q * FC, FC)
            @pl.when(sid < NS - 1)
            def _():
                pltpu.sync_copy(
                    h_hbm.at[pl.ds(row0, rows_t), pl.ds(col0, FC)],
                    acc.at[pl.ds(row0, rows_t)],
                )

            @pl.when(sid == NS - 1)
            def _():
                pltpu.sync_copy(
                    h_hbm.at[pl.ds(row0, rem), pl.ds(col0, FC)],
                    acc.at[pl.ds(row0, rem)],
                )

            plsc.subcore_barrier()

            def body(j, carry):
                eoff = pl.multiple_of(sid * e_t + j * WIN, WIN)
                pltpu.sync_copy(m_hbm.at[q, pl.ds(eoff, WIN)], mv)
                pltpu.sync_copy(mv, acc.at[idx_v.at[j]], add=True)
                return carry

            lax.fori_loop(0, wins_t, body, 0)
            plsc.subcore_barrier()
            pltpu.sync_copy(
                acc.at[pl.ds(row0, rows_t)],
                out_hbm.at[pl.ds(row0, rows_t), pl.ds(col0, FC)],
            )
            plsc.subcore_barrier()

    return k(m, dst_w, h)


def _tc_messages(g, ea, we, be):
    e, f = g.shape
    nq = f // FC
    eb = 2000

    def body(g_ref, ea_ref, we_ref, be_ref, m_ref):
        eab = ea_ref[...].astype(jnp.bfloat16).astype(jnp.float32)
        web = we_ref[...].astype(jnp.bfloat16).astype(jnp.float32)
        emb = (
            eab[:, 0:1] * web[0:1, :]
            + eab[:, 1:2] * web[1:2, :]
            + eab[:, 2:3] * web[2:3, :]
        )
        m_ref[0] = jnp.maximum(g_ref[...] + emb + be_ref[...], 0.0)

    return pl.pallas_call(
        body,
        grid=(e // eb, nq),
        in_specs=[
            pl.BlockSpec((eb, FC), lambda i, q: (i, q)),
            pl.BlockSpec((eb, 3), lambda i, q: (i, 0)),
            pl.BlockSpec((3, FC), lambda i, q: (0, q)),
            pl.BlockSpec((1, FC), lambda i, q: (0, q)),
        ],
        out_specs=pl.BlockSpec((1, eb, FC), lambda i, q: (q, i, 0)),
        out_shape=jax.ShapeDtypeStruct((nq, e, FC), jnp.float32),
    )(g, ea, we, be)


def _tc_mlp(z, wa, ba, wb, bb, nb):
    n_rows, f = z.shape
    h = wa.shape[1]
    h2 = wb.shape[1]

    def body(z_ref, wa_ref, ba_ref, wb_ref, bb_ref, h_ref):
        bf = jnp.bfloat16
        t = jnp.dot(z_ref[...].astype(bf), wa_ref[...].astype(bf),
                    preferred_element_type=jnp.float32)
        t = jnp.maximum(t + ba_ref[...], 0.0)
        out = jnp.dot(t.astype(bf), wb_ref[...].astype(bf),
                      preferred_element_type=jnp.float32)
        h_ref[...] = jnp.maximum(out + bb_ref[...], 0.0)

    return pl.pallas_call(
        body,
        grid=(n_rows // nb,),
        in_specs=[
            pl.BlockSpec((nb, f), lambda i: (i, 0)),
            pl.BlockSpec((f, h), lambda i: (0, 0)),
            pl.BlockSpec((1, h), lambda i: (0, 0)),
            pl.BlockSpec((h, h2), lambda i: (0, 0)),
            pl.BlockSpec((1, h2), lambda i: (0, 0)),
        ],
        out_specs=pl.BlockSpec((nb, h2), lambda i: (i, 0)),
        out_shape=jax.ShapeDtypeStruct((n_rows, h2), jnp.float32),
    )(z, wa, ba, wb, bb)


def _tc_pool_classify(h, batch_col, wc, bc):
    n = batch_col.shape[0]
    hh = h.shape[1]
    out_dim = wc.shape[1]
    nb = 1000
    nsteps = n // nb

    def body(h_ref, b_ref, wc_ref, bc_ref, out_ref, acc):
        i = pl.program_id(0)

        @pl.when(i == 0)
        def _():
            acc[...] = jnp.zeros_like(acc)

        ids = b_ref[...]
        onehot = (
            ids == lax.broadcasted_iota(jnp.int32, (nb, POOL_B), 1)
        ).astype(jnp.float32)
        acc[...] += lax.dot_general(
            onehot,
            h_ref[...],
            dimension_numbers=(((0,), (0,)), ((), ())),
            preferred_element_type=jnp.float32,
            precision=lax.Precision.HIGHEST,
        )

        @pl.when(i == nsteps - 1)
        def _():
            pooled = jnp.maximum(acc[...], 0.0)
            logits = (
                jnp.dot(pooled.astype(jnp.bfloat16),
                        wc_ref[...].astype(jnp.bfloat16),
                        preferred_element_type=jnp.float32)
                + bc_ref[...]
            )
            out_ref[...] = 1.0 / (1.0 + jnp.exp(-logits))

    return pl.pallas_call(
        body,
        grid=(nsteps,),
        in_specs=[
            pl.BlockSpec((nb, hh), lambda i: (i, 0)),
            pl.BlockSpec((nb, 1), lambda i: (i, 0)),
            pl.BlockSpec((hh, out_dim), lambda i: (0, 0)),
            pl.BlockSpec((1, out_dim), lambda i: (0, 0)),
        ],
        out_specs=pl.BlockSpec((POOL_B, out_dim), lambda i: (0, 0)),
        out_shape=jax.ShapeDtypeStruct((POOL_B, out_dim), jnp.float32),
        scratch_shapes=[pltpu.VMEM((POOL_B, hh), jnp.float32)],
    )(h, batch_col, wc, bc)


def kernel(x, edge_index, edge_attr, batch, We1, be1, W1a, b1a, W1b, b1b,
           We2, be2, W2a, b2a, W2b, b2b, Wc, bc):
    n, f_in = x.shape
    e = edge_index.shape[1]
    h_dim = W1a.shape[1]
    align = 8 * NS
    np_rows = ((n + align - 1) // align) * align
    src = edge_index[0]
    dst_w = edge_index[1].reshape(NS, e // (NS * WIN), WIN)
    mlp_nb = np_rows // 8

    g1 = _sc_gather(x, src)
    m1 = _tc_messages(g1, edge_attr, We1, be1.reshape(1, f_in))
    z1 = _sc_scatter_add(m1, dst_w, x, np_rows)
    h1 = _tc_mlp(z1, W1a, b1a.reshape(1, h_dim), W1b,
                 b1b.reshape(1, h_dim), mlp_nb)

    g2 = _sc_gather(h1, src)
    m2 = _tc_messages(g2, edge_attr, We2, be2.reshape(1, h_dim))
    z2 = _sc_scatter_add(m2, dst_w, h1, np_rows)
    h2 = _tc_mlp(z2, W2a, b2a.reshape(1, h_dim), W2b,
                 b2b.reshape(1, h_dim), mlp_nb)

    return _tc_pool_classify(h2, batch.reshape(n, 1), Wc, bc.reshape(1, -1))


# speedup vs baseline: 2.1067x; 1.3349x over previous
import functools

import jax
import jax.numpy as jnp
from jax import lax
from jax.experimental import pallas as pl
from jax.experimental.pallas import tpu as pltpu
from jax.experimental.pallas import tpu_sc as plsc

NC, NS = 2, 16
NW = NC * NS
FC = 128
WIN = 40
POOL_B = 64

_MESH = dict(core_axis_name="c", subcore_axis_name="s",
             num_cores=NC, num_subcores=NS)


def _sc_gather(table, idx):
    n, f = table.shape
    e = idx.shape[0]
    per_w = e // NW
    nwin = per_w // WIN

    @functools.partial(
        pl.kernel,
        mesh=plsc.VectorSubcoreMesh(**_MESH),
        out_type=jax.ShapeDtypeStruct((e, f), jnp.float32),
        scratch_types=[
            pltpu.VMEM((per_w,), jnp.int32),
            pltpu.VMEM((2, WIN, f), jnp.float32),
            pltpu.SemaphoreType.DMA,
            pltpu.SemaphoreType.DMA,
            pltpu.SemaphoreType.DMA,
            pltpu.SemaphoreType.DMA,
        ],
    )
    def k(table_hbm, idx_hbm, out_hbm, idx_v, rows_v, sg0, sg1, so0, so1):
        wid = lax.axis_index("s") * NC + lax.axis_index("c")
        base = pl.multiple_of(wid * per_w, per_w)
        pltpu.sync_copy(idx_hbm.at[pl.ds(base, per_w)], idx_v)

        def g_copy(w, b, sem):
            off = pl.multiple_of(w * WIN, WIN)
            return pltpu.make_async_copy(
                table_hbm.at[idx_v.at[pl.ds(off, WIN)]], rows_v.at[b], sem
            )

        def o_copy(w, b, sem):
            off = pl.multiple_of(base + w * WIN, WIN)
            return pltpu.make_async_copy(
                rows_v.at[b], out_hbm.at[pl.ds(off, WIN)], sem
            )

        g_copy(0, 0, sg0).start()

        def body(i, carry):
            w0 = 2 * i
            g_copy(w0, 0, sg0).wait()
            o_copy(w0, 0, so0).start()
            g_copy(w0 + 1, 1, sg1).start()
            o_copy(w0, 0, so0).wait()
            g_copy(w0 + 2, 0, sg0).start()
            g_copy(w0 + 1, 1, sg1).wait()
            o_copy(w0 + 1, 1, so1).start()
            o_copy(w0 + 1, 1, so1).wait()
            return carry

        lax.fori_loop(0, (nwin - 1) // 2, body, 0)
        g_copy(nwin - 1, 0, sg0).wait()
        o_copy(nwin - 1, 0, so0).start()
        o_copy(nwin - 1, 0, so0).wait()

    return k(table, idx)


def _sc_scatter_add(m, dst_w, h, np_rows):
    nq, e, _ = m.shape
    n, f = h.shape
    rows_t = np_rows // NS
    rem = n - (NS - 1) * rows_t
    e_t = e // NS
    wins_t = e_t // WIN
    passes = nq // NC

    @functools.partial(
        pl.kernel,
        mesh=plsc.VectorSubcoreMesh(**_MESH),
        out_type=jax.ShapeDtypeStruct((np_rows, nq * FC), jnp.float32),
        scratch_types=[
            pltpu.VMEM((wins_t, WIN), jnp.int32),
            pltpu.VMEM((2, WIN, FC), jnp.float32),
            pltpu.VMEM_SHARED((np_rows, FC), jnp.float32),
            pltpu.SemaphoreType.DMA,
            pltpu.SemaphoreType.DMA,
            pltpu.SemaphoreType.DMA,
            pltpu.SemaphoreType.DMA,
        ],
    )
    def k(m_hbm, dst_hbm, h_hbm, out_hbm, idx_v, mv, acc, sm0, sm1, ss0, ss1):
        cid = lax.axis_index("c")
        sid = lax.axis_index("s")
        row0 = pl.multiple_of(sid * rows_t, rows_t)
        pltpu.sync_copy(dst_hbm.at[sid], idx_v)

        for p in range(passes):
            q = p * NC + cid
            col0 = pl.multiple_of(q * FC, FC)
            @pl.when(sid < NS - 1)
            def _():
                pltpu.sync_copy(
                    h_hbm.at[pl.ds(row0, rows_t), pl.ds(col0, FC)],
                    acc.at[pl.ds(row0, rows_t)],
                )

            @pl.when(sid == NS - 1)
            def _():
                pltpu.sync_copy(
                    h_hbm.at[pl.ds(row0, rem), pl.ds(col0, FC)],
                    acc.at[pl.ds(row0, rem)],
                )

            plsc.subcore_barrier()

            def m_copy(w, b, sem):
                eoff = pl.multiple_of(sid * e_t + w * WIN, WIN)
                return pltpu.make_async_copy(
                    m_hbm.at[q, pl.ds(eoff, WIN)], mv.at[b], sem
                )

            def s_copy(w, b, sem):
                return pltpu.make_async_copy(mv.at[b], acc.at[idx_v.at[w]], sem)

            def s_add(w, b, sem):
                pltpu.async_copy(mv.at[b], acc.at[idx_v.at[w]], sem, add=True)

            m_copy(0, 0, sm0).start()
            last = wins_t // 2 - 1

            def body(i, carry):
                w0 = 2 * i
                m_copy(w0, 0, sm0).wait()
                s_add(w0, 0, ss0)
                m_copy(w0 + 1, 1, sm1).start()
                s_copy(w0, 0, ss0).wait()

                @pl.when(i < last)
                def _():
                    m_copy(w0 + 2, 0, sm0).start()

                m_copy(w0 + 1, 1, sm1).wait()
                s_add(w0 + 1, 1, ss1)
                s_copy(w0 + 1, 1, ss1).wait()
                return carry

            lax.fori_loop(0, wins_t // 2, body, 0)
            plsc.subcore_barrier()
            pltpu.sync_copy(
                acc.at[pl.ds(row0, rows_t)],
                out_hbm.at[pl.ds(row0, rows_t), pl.ds(col0, FC)],
            )
            plsc.subcore_barrier()

    return k(m, dst_w, h)


def _tc_messages(g, ea, we, be):
    e, f = g.shape
    nq = f // FC
    eb = 2000

    def body(g_ref, ea_ref, we_ref, be_ref, m_ref):
        eab = ea_ref[...].astype(jnp.bfloat16).astype(jnp.float32)
        web = we_ref[...].astype(jnp.bfloat16).astype(jnp.float32)
        emb = (
            eab[:, 0:1] * web[0:1, :]
            + eab[:, 1:2] * web[1:2, :]
            + eab[:, 2:3] * web[2:3, :]
        )
        m_ref[0] = jnp.maximum(g_ref[...] + emb + be_ref[...], 0.0)

    return pl.pallas_call(
        body,
        grid=(e // eb, nq),
        in_specs=[
            pl.BlockSpec((eb, FC), lambda i, q: (i, q)),
            pl.BlockSpec((eb, 3), lambda i, q: (i, 0)),
            pl.BlockSpec((3, FC), lambda i, q: (0, q)),
            pl.BlockSpec((1, FC), lambda i, q: (0, q)),
        ],
        out_specs=pl.BlockSpec((1, eb, FC), lambda i, q: (q, i, 0)),
        out_shape=jax.ShapeDtypeStruct((nq, e, FC), jnp.float32),
    )(g, ea, we, be)


def _tc_mlp(z, wa, ba, wb, bb, nb):
    n_rows, f = z.shape
    h = wa.shape[1]
    h2 = wb.shape[1]

    def body(z_ref, wa_ref, ba_ref, wb_ref, bb_ref, h_ref):
        bf = jnp.bfloat16
        t = jnp.dot(z_ref[...].astype(bf), wa_ref[...].astype(bf),
                    preferred_element_type=jnp.float32)
        t = jnp.maximum(t + ba_ref[...], 0.0)
        out = jnp.dot(t.astype(bf), wb_ref[...].astype(bf),
                      preferred_element_type=jnp.float32)
        h_ref[...] = jnp.maximum(out + bb_ref[...], 0.0)

    return pl.pallas_call(
        body,
        grid=(n_rows // nb,),
        in_specs=[
            pl.BlockSpec((nb, f), lambda i: (i, 0)),
            pl.BlockSpec((f, h), lambda i: (0, 0)),
            pl.BlockSpec((1, h), lambda i: (0, 0)),
            pl.BlockSpec((h, h2), lambda i: (0, 0)),
            pl.BlockSpec((1, h2), lambda i: (0, 0)),
        ],
        out_specs=pl.BlockSpec((nb, h2), lambda i: (i, 0)),
        out_shape=jax.ShapeDtypeStruct((n_rows, h2), jnp.float32),
    )(z, wa, ba, wb, bb)


def _tc_pool_classify(h, batch_col, wc, bc):
    n = batch_col.shape[0]
    hh = h.shape[1]
    out_dim = wc.shape[1]
    nb = 1000
    nsteps = n // nb

    def body(h_ref, b_ref, wc_ref, bc_ref, out_ref, acc):
        i = pl.program_id(0)

        @pl.when(i == 0)
        def _():
            acc[...] = jnp.zeros_like(acc)

        ids = b_ref[...]
        onehot = (
            ids == lax.broadcasted_iota(jnp.int32, (nb, POOL_B), 1)
        ).astype(jnp.float32)
        acc[...] += lax.dot_general(
            onehot,
            h_ref[...],
            dimension_numbers=(((0,), (0,)), ((), ())),
            preferred_element_type=jnp.float32,
            precision=lax.Precision.HIGHEST,
        )

        @pl.when(i == nsteps - 1)
        def _():
            pooled = jnp.maximum(acc[...], 0.0)
            logits = (
                jnp.dot(pooled.astype(jnp.bfloat16),
                        wc_ref[...].astype(jnp.bfloat16),
                        preferred_element_type=jnp.float32)
                + bc_ref[...]
            )
            out_ref[...] = 1.0 / (1.0 + jnp.exp(-logits))

    return pl.pallas_call(
        body,
        grid=(nsteps,),
        in_specs=[
            pl.BlockSpec((nb, hh), lambda i: (i, 0)),
            pl.BlockSpec((nb, 1), lambda i: (i, 0)),
            pl.BlockSpec((hh, out_dim), lambda i: (0, 0)),
            pl.BlockSpec((1, out_dim), lambda i: (0, 0)),
        ],
        out_specs=pl.BlockSpec((POOL_B, out_dim), lambda i: (0, 0)),
        out_shape=jax.ShapeDtypeStruct((POOL_B, out_dim), jnp.float32),
        scratch_shapes=[pltpu.VMEM((POOL_B, hh), jnp.float32)],
    )(h, batch_col, wc, bc)


def kernel(x, edge_index, edge_attr, batch, We1, be1, W1a, b1a, W1b, b1b,
           We2, be2, W2a, b2a, W2b, b2b, Wc, bc):
    n, f_in = x.shape
    e = edge_index.shape[1]
    h_dim = W1a.shape[1]
    align = 8 * NS
    np_rows = ((n + align - 1) // align) * align
    src = edge_index[0]
    dst_w = edge_index[1].reshape(NS, e // (NS * WIN), WIN)
    mlp_nb = np_rows // 8

    g1 = _sc_gather(x, src)
    m1 = _tc_messages(g1, edge_attr, We1, be1.reshape(1, f_in))
    z1 = _sc_scatter_add(m1, dst_w, x, np_rows)
    h1 = _tc_mlp(z1, W1a, b1a.reshape(1, h_dim), W1b,
                 b1b.reshape(1, h_dim), mlp_nb)

    g2 = _sc_gather(h1, src)
    m2 = _tc_messages(g2, edge_attr, We2, be2.reshape(1, h_dim))
    z2 = _sc_scatter_add(m2, dst_w, h1, np_rows)
    h2 = _tc_mlp(z2, W2a, b2a.reshape(1, h_dim), W2b,
                 b2b.reshape(1, h_dim), mlp_nb)

    return _tc_pool_classify(h2, batch.reshape(n, 1), Wc, bc.reshape(1, -1))


# speedup vs baseline: 2.1177x; 1.0052x over previous
import functools

import jax
import jax.numpy as jnp
from jax import lax
from jax.experimental import pallas as pl
from jax.experimental.pallas import tpu as pltpu
from jax.experimental.pallas import tpu_sc as plsc

NC, NS = 2, 16
NW = NC * NS
FC = 128
WIN = 40
GW = 1
POOL_B = 64

_MESH = dict(core_axis_name="c", subcore_axis_name="s",
             num_cores=NC, num_subcores=NS)


def _sc_gather(table, idx):
    n, f = table.shape
    e = idx.shape[0]
    per_w = e // NW
    nwin = per_w // WIN

    @functools.partial(
        pl.kernel,
        mesh=plsc.VectorSubcoreMesh(**_MESH),
        out_type=jax.ShapeDtypeStruct((e, f), jnp.float32),
        scratch_types=[
            pltpu.VMEM((per_w,), jnp.int32),
            pltpu.VMEM((4, WIN, f), jnp.float32),
            pltpu.SemaphoreType.DMA,
            pltpu.SemaphoreType.DMA,
        ],
    )
    def k(table_hbm, idx_hbm, out_hbm, idx_v, rows_v, sg, so):
        wid = lax.axis_index("s") * NC + lax.axis_index("c")
        base = pl.multiple_of(wid * per_w, per_w)
        pltpu.sync_copy(idx_hbm.at[pl.ds(base, per_w)], idx_v)

        def g_copy(w, b):
            off = pl.multiple_of(w * WIN, WIN)
            return pltpu.make_async_copy(
                table_hbm.at[idx_v.at[pl.ds(off, WIN)]], rows_v.at[b], sg
            )

        def o_copy(w, b):
            off = pl.multiple_of(base + w * WIN, WIN)
            return pltpu.make_async_copy(
                rows_v.at[b], out_hbm.at[pl.ds(off, WIN)], so
            )

        g_copy(0, 0).start()
        g_copy(1, 1).start()

        def body(w, carry):
            b = w % 4
            g_copy(w, b).wait()
            o_copy(w, b).start()

            @pl.when(w >= 2)
            def _():
                o_copy(w - 2, (w - 2) % 4).wait()

            @pl.when(w + 2 < nwin)
            def _():
                g_copy(w + 2, (w + 2) % 4).start()

            return carry

        lax.fori_loop(0, nwin, body, 0)
        o_copy(nwin - 2, (nwin - 2) % 4).wait()
        o_copy(nwin - 1, (nwin - 1) % 4).wait()

    return k(table, idx)


def _sc_scatter_add(m, dst_w, h, np_rows):
    nq, e, _ = m.shape
    n, f = h.shape
    rows_t = np_rows // NS
    rem = n - (NS - 1) * rows_t
    e_t = e // NS
    wins_t = e_t // WIN
    passes = nq // NC

    @functools.partial(
        pl.kernel,
        mesh=plsc.VectorSubcoreMesh(**_MESH),
        out_type=jax.ShapeDtypeStruct((np_rows, nq * FC), jnp.float32),
        scratch_types=[
            pltpu.VMEM((wins_t, WIN), jnp.int32),
            pltpu.VMEM((2, GW * WIN, FC), jnp.float32),
            pltpu.VMEM_SHARED((np_rows, FC), jnp.float32),
            pltpu.SemaphoreType.DMA,
            pltpu.SemaphoreType.DMA,
            pltpu.SemaphoreType.DMA,
            pltpu.SemaphoreType.DMA,
        ],
    )
    def k(m_hbm, dst_hbm, h_hbm, out_hbm, idx_v, mv, acc, sm0, sm1, ss0, ss1):
        cid = lax.axis_index("c")
        sid = lax.axis_index("s")
        row0 = pl.multiple_of(sid * rows_t, rows_t)
        pltpu.sync_copy(dst_hbm.at[sid], idx_v)

        for p in range(passes):
            q = p * NC + cid
            col0 = pl.multiple_of(q * FC, FC)
            @pl.when(sid < NS - 1)
            def _():
                pltpu.sync_copy(
                    h_hbm.at[pl.ds(row0, rows_t), pl.ds(col0, FC)],
                    acc.at[pl.ds(row0, rows_t)],
                )

            @pl.when(sid == NS - 1)
            def _():
                pltpu.sync_copy(
                    h_hbm.at[pl.ds(row0, rem), pl.ds(col0, FC)],
                    acc.at[pl.ds(row0, rem)],
                )

            plsc.subcore_barrier()

            def m_copy(g, b, sem):
                eoff = pl.multiple_of(sid * e_t + g * (GW * WIN), WIN)
                return pltpu.make_async_copy(
                    m_hbm.at[q, pl.ds(eoff, GW * WIN)], mv.at[b], sem
                )

            def s_fire(g, b, sem):
                for j in range(GW):
                    pltpu.async_copy(
                        mv.at[b, pl.ds(j * WIN, WIN)],
                        acc.at[idx_v.at[g * GW + j]],
                        sem,
                        add=True,
                    )

            def s_drain(g, b, sem):
                for j in range(GW):
                    pltpu.make_async_copy(
                        mv.at[b, pl.ds(j * WIN, WIN)],
                        acc.at[idx_v.at[g * GW + j]],
                        sem,
                    ).wait()

            ngroups = wins_t // GW
            m_copy(0, 0, sm0).start()

            def body(i, carry):
                g0 = 2 * i
                m_copy(g0, 0, sm0).wait()
                s_fire(g0, 0, ss0)
                m_copy(g0 + 1, 1, sm1).start()
                s_drain(g0, 0, ss0)

                @pl.when(i < ngroups // 2 - 1)
                def _():
                    m_copy(g0 + 2, 0, sm0).start()

                m_copy(g0 + 1, 1, sm1).wait()
                s_fire(g0 + 1, 1, ss1)
                s_drain(g0 + 1, 1, ss1)
                return carry

            lax.fori_loop(0, ngroups // 2, body, 0)
            plsc.subcore_barrier()
            pltpu.sync_copy(
                acc.at[pl.ds(row0, rows_t)],
                out_hbm.at[pl.ds(row0, rows_t), pl.ds(col0, FC)],
            )
            plsc.subcore_barrier()

    return k(m, dst_w, h)


def _tc_messages(g, ea, we, be):
    e, f = g.shape
    nq = f // FC
    eb = 2000

    def body(g_ref, ea_ref, we_ref, be_ref, m_ref):
        eab = ea_ref[...].astype(jnp.bfloat16).astype(jnp.float32)
        web = we_ref[...].astype(jnp.bfloat16).astype(jnp.float32)
        emb = (
            eab[:, 0:1] * web[0:1, :]
            + eab[:, 1:2] * web[1:2, :]
            + eab[:, 2:3] * web[2:3, :]
        )
        m_ref[0] = jnp.maximum(g_ref[...] + emb + be_ref[...], 0.0)

    return pl.pallas_call(
        body,
        grid=(e // eb, nq),
        in_specs=[
            pl.BlockSpec((eb, FC), lambda i, q: (i, q)),
            pl.BlockSpec((eb, 3), lambda i, q: (i, 0)),
            pl.BlockSpec((3, FC), lambda i, q: (0, q)),
            pl.BlockSpec((1, FC), lambda i, q: (0, q)),
        ],
        out_specs=pl.BlockSpec((1, eb, FC), lambda i, q: (q, i, 0)),
        out_shape=jax.ShapeDtypeStruct((nq, e, FC), jnp.float32),
    )(g, ea, we, be)


def _tc_mlp(z, wa, ba, wb, bb, nb):
    n_rows, f = z.shape
    h = wa.shape[1]
    h2 = wb.shape[1]

    def body(z_ref, wa_ref, ba_ref, wb_ref, bb_ref, h_ref):
        bf = jnp.bfloat16
        t = jnp.dot(z_ref[...].astype(bf), wa_ref[...].astype(bf),
                    preferred_element_type=jnp.float32)
        t = jnp.maximum(t + ba_ref[...], 0.0)
        out = jnp.dot(t.astype(bf), wb_ref[...].astype(bf),
                      preferred_element_type=jnp.float32)
        h_ref[...] = jnp.maximum(out + bb_ref[...], 0.0)

    return pl.pallas_call(
        body,
        grid=(n_rows // nb,),
        in_specs=[
            pl.BlockSpec((nb, f), lambda i: (i, 0)),
            pl.BlockSpec((f, h), lambda i: (0, 0)),
            pl.BlockSpec((1, h), lambda i: (0, 0)),
            pl.BlockSpec((h, h2), lambda i: (0, 0)),
            pl.BlockSpec((1, h2), lambda i: (0, 0)),
        ],
        out_specs=pl.BlockSpec((nb, h2), lambda i: (i, 0)),
        out_shape=jax.ShapeDtypeStruct((n_rows, h2), jnp.float32),
    )(z, wa, ba, wb, bb)


def _tc_pool_classify(h, batch_col, wc, bc):
    n = batch_col.shape[0]
    hh = h.shape[1]
    out_dim = wc.shape[1]
    nb = 1000
    nsteps = n // nb

    def body(h_ref, b_ref, wc_ref, bc_ref, out_ref, acc):
        i = pl.program_id(0)

        @pl.when(i == 0)
        def _():
            acc[...] = jnp.zeros_like(acc)

        ids = b_ref[...]
        onehot = (
            ids == lax.broadcasted_iota(jnp.int32, (nb, POOL_B), 1)
        ).astype(jnp.float32)
        acc[...] += lax.dot_general(
            onehot,
            h_ref[...],
            dimension_numbers=(((0,), (0,)), ((), ())),
            preferred_element_type=jnp.float32,
            precision=lax.Precision.HIGHEST,
        )

        @pl.when(i == nsteps - 1)
        def _():
            pooled = jnp.maximum(acc[...], 0.0)
            logits = (
                jnp.dot(pooled.astype(jnp.bfloat16),
                        wc_ref[...].astype(jnp.bfloat16),
                        preferred_element_type=jnp.float32)
                + bc_ref[...]
            )
            out_ref[...] = 1.0 / (1.0 + jnp.exp(-logits))

    return pl.pallas_call(
        body,
        grid=(nsteps,),
        in_specs=[
            pl.BlockSpec((nb, hh), lambda i: (i, 0)),
            pl.BlockSpec((nb, 1), lambda i: (i, 0)),
            pl.BlockSpec((hh, out_dim), lambda i: (0, 0)),
            pl.BlockSpec((1, out_dim), lambda i: (0, 0)),
        ],
        out_specs=pl.BlockSpec((POOL_B, out_dim), lambda i: (0, 0)),
        out_shape=jax.ShapeDtypeStruct((POOL_B, out_dim), jnp.float32),
        scratch_shapes=[pltpu.VMEM((POOL_B, hh), jnp.float32)],
    )(h, batch_col, wc, bc)


def kernel(x, edge_index, edge_attr, batch, We1, be1, W1a, b1a, W1b, b1b,
           We2, be2, W2a, b2a, W2b, b2b, Wc, bc):
    n, f_in = x.shape
    e = edge_index.shape[1]
    h_dim = W1a.shape[1]
    align = 8 * NS
    np_rows = ((n + align - 1) // align) * align
    src = edge_index[0]
    dst_w = edge_index[1].reshape(NS, e // (NS * WIN), WIN)
    mlp_nb = np_rows // 8

    g1 = _sc_gather(x, src)
    m1 = _tc_messages(g1, edge_attr, We1, be1.reshape(1, f_in))
    z1 = _sc_scatter_add(m1, dst_w, x, np_rows)
    h1 = _tc_mlp(z1, W1a, b1a.reshape(1, h_dim), W1b,
                 b1b.reshape(1, h_dim), mlp_nb)

    g2 = _sc_gather(h1, src)
    m2 = _tc_messages(g2, edge_attr, We2, be2.reshape(1, h_dim))
    z2 = _sc_scatter_add(m2, dst_w, h1, np_rows)
    h2 = _tc_mlp(z2, W2a, b2a.reshape(1, h_dim), W2b,
                 b2b.reshape(1, h_dim), mlp_nb)

    return _tc_pool_classify(h2, batch.reshape(n, 1), Wc, bc.reshape(1, -1))


# speedup vs baseline: 2.1210x; 1.0015x over previous
import functools

import jax
import jax.numpy as jnp
from jax import lax
from jax.experimental import pallas as pl
from jax.experimental.pallas import tpu as pltpu
from jax.experimental.pallas import tpu_sc as plsc

NC, NS = 2, 16
NW = NC * NS
FC = 128
WIN = 40
GW = 1
POOL_B = 64

_MESH = dict(core_axis_name="c", subcore_axis_name="s",
             num_cores=NC, num_subcores=NS)


def _sc_gather(table, idx):
    n, f = table.shape
    e = idx.shape[0]
    per_w = e // NW
    nwin = per_w // WIN

    @functools.partial(
        pl.kernel,
        mesh=plsc.VectorSubcoreMesh(**_MESH),
        out_type=jax.ShapeDtypeStruct((e, f), jnp.float32),
        scratch_types=[
            pltpu.VMEM((per_w,), jnp.int32),
            pltpu.VMEM((5, WIN, f), jnp.float32),
            pltpu.SemaphoreType.DMA,
            pltpu.SemaphoreType.DMA,
        ],
    )
    def k(table_hbm, idx_hbm, out_hbm, idx_v, rows_v, sg, so):
        wid = lax.axis_index("s") * NC + lax.axis_index("c")
        base = pl.multiple_of(wid * per_w, per_w)
        pltpu.sync_copy(idx_hbm.at[pl.ds(base, per_w)], idx_v)

        def g_copy(w, b):
            off = pl.multiple_of(w * WIN, WIN)
            return pltpu.make_async_copy(
                table_hbm.at[idx_v.at[pl.ds(off, WIN)]], rows_v.at[b], sg
            )

        def o_copy(w, b):
            off = pl.multiple_of(base + w * WIN, WIN)
            return pltpu.make_async_copy(
                rows_v.at[b], out_hbm.at[pl.ds(off, WIN)], so
            )

        g_copy(0, 0).start()
        g_copy(1, 1).start()
        g_copy(2, 2).start()

        def body(w, carry):
            b = w % 5
            g_copy(w, b).wait()
            o_copy(w, b).start()

            @pl.when(w >= 2)
            def _():
                o_copy(w - 2, (w - 2) % 5).wait()

            @pl.when(w + 3 < nwin)
            def _():
                g_copy(w + 3, (w + 3) % 5).start()

            return carry

        lax.fori_loop(0, nwin, body, 0)
        o_copy(nwin - 2, (nwin - 2) % 5).wait()
        o_copy(nwin - 1, (nwin - 1) % 5).wait()

    return k(table, idx)


def _sc_scatter_add(m, dst_w, h, np_rows):
    nq, e, _ = m.shape
    n, f = h.shape
    rows_t = np_rows // NS
    rem = n - (NS - 1) * rows_t
    e_t = e // NS
    wins_t = e_t // WIN
    passes = nq // NC

    @functools.partial(
        pl.kernel,
        mesh=plsc.VectorSubcoreMesh(**_MESH),
        out_type=jax.ShapeDtypeStruct((np_rows, nq * FC), jnp.float32),
        scratch_types=[
            pltpu.VMEM((wins_t, WIN), jnp.int32),
            pltpu.VMEM((2, GW * WIN, FC), jnp.float32),
            pltpu.VMEM_SHARED((np_rows, FC), jnp.float32),
            pltpu.SemaphoreType.DMA,
            pltpu.SemaphoreType.DMA,
            pltpu.SemaphoreType.DMA,
            pltpu.SemaphoreType.DMA,
        ],
    )
    def k(m_hbm, dst_hbm, h_hbm, out_hbm, idx_v, mv, acc, sm0, sm1, ss0, ss1):
        cid = lax.axis_index("c")
        sid = lax.axis_index("s")
        row0 = pl.multiple_of(sid * rows_t, rows_t)
        pltpu.sync_copy(dst_hbm.at[sid], idx_v)

        for p in range(passes):
            q = p * NC + cid
            col0 = pl.multiple_of(q * FC, FC)
            @pl.when(sid < NS - 1)
            def _():
                pltpu.sync_copy(
                    h_hbm.at[pl.ds(row0, rows_t), pl.ds(col0, FC)],
                    acc.at[pl.ds(row0, rows_t)],
                )

            @pl.when(sid == NS - 1)
            def _():
                pltpu.sync_copy(
                    h_hbm.at[pl.ds(row0, rem), pl.ds(col0, FC)],
                    acc.at[pl.ds(row0, rem)],
                )

            plsc.subcore_barrier()

            def m_copy(g, b, sem):
                eoff = pl.multiple_of(sid * e_t + g * (GW * WIN), WIN)
                return pltpu.make_async_copy(
                    m_hbm.at[q, pl.ds(eoff, GW * WIN)], mv.at[b], sem
                )

            def s_fire(g, b, sem):
                for j in range(GW):
                    pltpu.async_copy(
                        mv.at[b, pl.ds(j * WIN, WIN)],
                        acc.at[idx_v.at[g * GW + j]],
                        sem,
                        add=True,
                    )

            def s_drain(g, b, sem):
                for j in range(GW):
                    pltpu.make_async_copy(
                        mv.at[b, pl.ds(j * WIN, WIN)],
                        acc.at[idx_v.at[g * GW + j]],
                        sem,
                    ).wait()

            ngroups = wins_t // GW
            m_copy(0, 0, sm0).start()

            def body(i, carry):
                g0 = 2 * i
                m_copy(g0, 0, sm0).wait()
                s_fire(g0, 0, ss0)
                m_copy(g0 + 1, 1, sm1).start()
                s_drain(g0, 0, ss0)

                @pl.when(i < ngroups // 2 - 1)
                def _():
                    m_copy(g0 + 2, 0, sm0).start()

                m_copy(g0 + 1, 1, sm1).wait()
                s_fire(g0 + 1, 1, ss1)
                s_drain(g0 + 1, 1, ss1)
                return carry

            lax.fori_loop(0, ngroups // 2, body, 0)
            plsc.subcore_barrier()
            pltpu.sync_copy(
                acc.at[pl.ds(row0, rows_t)],
                out_hbm.at[pl.ds(row0, rows_t), pl.ds(col0, FC)],
            )
            plsc.subcore_barrier()

    return k(m, dst_w, h)


def _tc_messages(g, ea, we, be):
    e, f = g.shape
    nq = f // FC
    eb = 2000

    def body(g_ref, ea_ref, we_ref, be_ref, m_ref):
        eab = ea_ref[...].astype(jnp.bfloat16).astype(jnp.float32)
        web = we_ref[...].astype(jnp.bfloat16).astype(jnp.float32)
        emb = (
            eab[:, 0:1] * web[0:1, :]
            + eab[:, 1:2] * web[1:2, :]
            + eab[:, 2:3] * web[2:3, :]
        )
        m_ref[0] = jnp.maximum(g_ref[...] + emb + be_ref[...], 0.0)

    return pl.pallas_call(
        body,
        grid=(e // eb, nq),
        in_specs=[
            pl.BlockSpec((eb, FC), lambda i, q: (i, q)),
            pl.BlockSpec((eb, 3), lambda i, q: (i, 0)),
            pl.BlockSpec((3, FC), lambda i, q: (0, q)),
            pl.BlockSpec((1, FC), lambda i, q: (0, q)),
        ],
        out_specs=pl.BlockSpec((1, eb, FC), lambda i, q: (q, i, 0)),
        out_shape=jax.ShapeDtypeStruct((nq, e, FC), jnp.float32),
    )(g, ea, we, be)


def _tc_mlp(z, wa, ba, wb, bb, nb):
    n_rows, f = z.shape
    h = wa.shape[1]
    h2 = wb.shape[1]

    def body(z_ref, wa_ref, ba_ref, wb_ref, bb_ref, h_ref):
        bf = jnp.bfloat16
        t = jnp.dot(z_ref[...].astype(bf), wa_ref[...].astype(bf),
                    preferred_element_type=jnp.float32)
        t = jnp.maximum(t + ba_ref[...], 0.0)
        out = jnp.dot(t.astype(bf), wb_ref[...].astype(bf),
                      preferred_element_type=jnp.float32)
        h_ref[...] = jnp.maximum(out + bb_ref[...], 0.0)

    return pl.pallas_call(
        body,
        grid=(n_rows // nb,),
        in_specs=[
            pl.BlockSpec((nb, f), lambda i: (i, 0)),
            pl.BlockSpec((f, h), lambda i: (0, 0)),
            pl.BlockSpec((1, h), lambda i: (0, 0)),
            pl.BlockSpec((h, h2), lambda i: (0, 0)),
            pl.BlockSpec((1, h2), lambda i: (0, 0)),
        ],
        out_specs=pl.BlockSpec((nb, h2), lambda i: (i, 0)),
        out_shape=jax.ShapeDtypeStruct((n_rows, h2), jnp.float32),
    )(z, wa, ba, wb, bb)


def _tc_pool_classify(h, batch_col, wc, bc):
    n = batch_col.shape[0]
    hh = h.shape[1]
    out_dim = wc.shape[1]
    nb = 1000
    nsteps = n // nb

    def body(h_ref, b_ref, wc_ref, bc_ref, out_ref, acc):
        i = pl.program_id(0)

        @pl.when(i == 0)
        def _():
            acc[...] = jnp.zeros_like(acc)

        ids = b_ref[...]
        onehot = (
            ids == lax.broadcasted_iota(jnp.int32, (nb, POOL_B), 1)
        ).astype(jnp.float32)
        acc[...] += lax.dot_general(
            onehot,
            h_ref[...],
            dimension_numbers=(((0,), (0,)), ((), ())),
            preferred_element_type=jnp.float32,
            precision=lax.Precision.HIGHEST,
        )

        @pl.when(i == nsteps - 1)
        def _():
            pooled = jnp.maximum(acc[...], 0.0)
            logits = (
                jnp.dot(pooled.astype(jnp.bfloat16),
                        wc_ref[...].astype(jnp.bfloat16),
                        preferred_element_type=jnp.float32)
                + bc_ref[...]
            )
            out_ref[...] = 1.0 / (1.0 + jnp.exp(-logits))

    return pl.pallas_call(
        body,
        grid=(nsteps,),
        in_specs=[
            pl.BlockSpec((nb, hh), lambda i: (i, 0)),
            pl.BlockSpec((nb, 1), lambda i: (i, 0)),
            pl.BlockSpec((hh, out_dim), lambda i: (0, 0)),
            pl.BlockSpec((1, out_dim), lambda i: (0, 0)),
        ],
        out_specs=pl.BlockSpec((POOL_B, out_dim), lambda i: (0, 0)),
        out_shape=jax.ShapeDtypeStruct((POOL_B, out_dim), jnp.float32),
        scratch_shapes=[pltpu.VMEM((POOL_B, hh), jnp.float32)],
    )(h, batch_col, wc, bc)


def kernel(x, edge_index, edge_attr, batch, We1, be1, W1a, b1a, W1b, b1b,
           We2, be2, W2a, b2a, W2b, b2b, Wc, bc):
    n, f_in = x.shape
    e = edge_index.shape[1]
    h_dim = W1a.shape[1]
    align = 8 * NS
    np_rows = ((n + align - 1) // align) * align
    src = edge_index[0]
    dst_w = edge_index[1].reshape(NS, e // (NS * WIN), WIN)
    mlp_nb = np_rows // 8

    g1 = _sc_gather(x, src)
    m1 = _tc_messages(g1, edge_attr, We1, be1.reshape(1, f_in))
    z1 = _sc_scatter_add(m1, dst_w, x, np_rows)
    h1 = _tc_mlp(z1, W1a, b1a.reshape(1, h_dim), W1b,
                 b1b.reshape(1, h_dim), mlp_nb)

    g2 = _sc_gather(h1, src)
    m2 = _tc_messages(g2, edge_attr, We2, be2.reshape(1, h_dim))
    z2 = _sc_scatter_add(m2, dst_w, h1, np_rows)
    h2 = _tc_mlp(z2, W2a, b2a.reshape(1, h_dim), W2b,
                 b2b.reshape(1, h_dim), mlp_nb)

    return _tc_pool_classify(h2, batch.reshape(n, 1), Wc, bc.reshape(1, -1))
